# baseline (speedup 1.0000x reference)
import functools

import jax
import jax.numpy as jnp
from jax import lax
from jax.experimental import pallas as pl
from jax.experimental.pallas import tpu as pltpu
from jax.experimental.pallas import tpu_sc as plsc

N = 10000
E = 320000
D = 128
NPAD = 10240
NC = 2
NS = 16
NW = NC * NS
EPT = E // NW
CH = 96
NCHUNK = -(-EPT // CH)
NCHP = NCHUNK + 2
SLAB = NCHP * CH
TRASH = N
RPT = NPAD // NS


def _sc_body(compute_deg, g_hbm, pk_hbm, acc_out, deg_out,
             acc_sp, deg_sp, packed, rows_a, rows_b, rows_c,
             sa_src, sa_dst, sb_src, sb_dst, sc_src, sc_dst,
             ones_v, zrow_v, ga, gb, gc, sa, sb, sc_, dsem):
    c = lax.axis_index("c")
    s = lax.axis_index("s")
    wid = c * NS + s

    pltpu.sync_copy(pk_hbm.at[wid], packed)

    def _zero_row(j, carry):
        for i in range(D // 16):
            rows_a[j, pl.ds(i * 16, 16)] = jnp.zeros((16,), jnp.float32)
        return carry
    lax.fori_loop(0, CH, _zero_row, 0, unroll=False)

    def _zrow16(i, carry):
        zrow_v[pl.ds(i * 16, 16)] = jnp.zeros((16,), jnp.float32)
        return carry
    lax.fori_loop(0, 128 // 16, _zrow16, 0, unroll=False)
    if compute_deg:
        def _ones16(i, carry):
            ones_v[pl.ds(i * 16, 16)] = jnp.ones((16,), jnp.float32)
            return carry
        lax.fori_loop(0, CH // 16, _ones16, 0, unroll=False)

    off = 0
    while off < RPT:
        nrows = min(CH, RPT - off)
        pltpu.sync_copy(rows_a.at[pl.ds(0, nrows)],
                        acc_sp.at[pl.ds(s * RPT + off, nrows)])
        off += nrows
    if compute_deg:
        for j in range(RPT // 128):
            pltpu.sync_copy(zrow_v,
                            deg_sp.at[pl.ds(s * RPT + j * 128, 128)])

    def unpack(k, st_src, st_dst):
        for i in range(CH // 16):
            w = packed[pl.ds(k * CH + i * 16, 16)]
            st_src[pl.ds(i * 16, 16)] = w & 0xFFFF
            st_dst[pl.ds(i * 16, 16)] = w >> 16

    def gather(st_src, buf, gsem):
        pltpu.async_copy(g_hbm.at[st_src], buf, gsem)

    def gather_wait(st_src, buf, gsem):
        pltpu.make_async_copy(g_hbm.at[st_src], buf, gsem).wait()

    def scat(st_dst, buf, ssem):
        pltpu.async_copy(buf, acc_sp.at[st_dst], ssem, add=True)
        if compute_deg:
            pltpu.async_copy(ones_v, deg_sp.at[st_dst], dsem, add=True)

    def scat_wait(st_dst, buf, ssem):
        pltpu.make_async_copy(buf, acc_sp.at[st_dst], ssem).wait()
        if compute_deg:
            pltpu.make_async_copy(ones_v, deg_sp.at[st_dst], dsem).wait()

    unpack(0, sa_src, sa_dst)
    unpack(1, sb_src, sb_dst)
    gather(sa_src, rows_a, ga)
    gather(sb_src, rows_b, gb)
    plsc.subcore_barrier()

    def body(t, carry):
        k0 = 3 * t
        gather_wait(sa_src, rows_a, ga)
        scat(sa_dst, rows_a, sa)

        @pl.when(t > 0)
        def _():
            scat_wait(sc_dst, rows_c, sc_)
        unpack(k0 + 2, sc_src, sc_dst)
        gather(sc_src, rows_c, gc)

        gather_wait(sb_src, rows_b, gb)
        scat(sb_dst, rows_b, sb)

        scat_wait(sa_dst, rows_a, sa)
        unpack(k0 + 3, sa_src, sa_dst)
        gather(sa_src, rows_a, ga)

        gather_wait(sc_src, rows_c, gc)
        scat(sc_dst, rows_c, sc_)

        scat_wait(sb_dst, rows_b, sb)
        unpack(k0 + 4, sb_src, sb_dst)
        gather(sb_src, rows_b, gb)
        return carry

    lax.fori_loop(0, NCHUNK // 3, body, 0, unroll=False)

    gather_wait(sa_src, rows_a, ga)
    gather_wait(sb_src, rows_b, gb)
    scat_wait(sc_dst, rows_c, sc_)
    plsc.subcore_barrier()

    pltpu.sync_copy(acc_sp.at[pl.ds(s * RPT, RPT)],
                    acc_out.at[pl.ds(c * NPAD + s * RPT, RPT)])
    if compute_deg:
        pltpu.sync_copy(deg_sp.at[pl.ds(s * RPT, RPT)],
                        deg_out.at[pl.ds(c * NPAD + s * RPT, RPT)])


def _make_sc(compute_deg):
    out_type = (jax.ShapeDtypeStruct((NC * NPAD, D), jnp.float32),
                jax.ShapeDtypeStruct((NC * NPAD,), jnp.float32))
    scratch = [
        pltpu.VMEM_SHARED((NPAD, D), jnp.float32),
        pltpu.VMEM_SHARED((NPAD,), jnp.float32),
        pltpu.VMEM((SLAB,), jnp.int32),
        pltpu.VMEM((CH, D), jnp.float32),
        pltpu.VMEM((CH, D), jnp.float32),
        pltpu.VMEM((CH, D), jnp.float32),
        pltpu.VMEM((CH,), jnp.int32),
        pltpu.VMEM((CH,), jnp.int32),
        pltpu.VMEM((CH,), jnp.int32),
        pltpu.VMEM((CH,), jnp.int32),
        pltpu.VMEM((CH,), jnp.int32),
        pltpu.VMEM((CH,), jnp.int32),
        pltpu.VMEM((CH,), jnp.float32),
        pltpu.VMEM((128,), jnp.float32),
        pltpu.SemaphoreType.DMA,
        pltpu.SemaphoreType.DMA,
        pltpu.SemaphoreType.DMA,
        pltpu.SemaphoreType.DMA,
        pltpu.SemaphoreType.DMA,
        pltpu.SemaphoreType.DMA,
        pltpu.SemaphoreType.DMA,
    ]
    mesh = plsc.VectorSubcoreMesh(core_axis_name="c", subcore_axis_name="s",
                                  num_cores=NC, num_subcores=NS)
    return pl.kernel(functools.partial(_sc_body, compute_deg),
                     out_type=out_type, mesh=mesh, scratch_types=scratch)


_TCB = 1000
_TCG = N // _TCB


def _tc1_body(h_ref, ws_ref, wn_ref, b_ref, s_ref, g_ref):
    hb = h_ref[...]
    s_ref[...] = jnp.dot(hb, ws_ref[...],
                         preferred_element_type=jnp.float32) + b_ref[...]
    g_ref[...] = jnp.dot(hb, wn_ref[...], preferred_element_type=jnp.float32)


def _tc2_body(s1_ref, p_ref, dg_ref, ws_ref, wn_ref, b_ref, s_ref, g_ref):
    psum = p_ref[0] + p_ref[1]
    deg = dg_ref[0] + dg_ref[1]
    inv = 1.0 / jnp.maximum(deg, 1.0)
    x = jnp.maximum(s1_ref[...] + psum * inv, 0.0)
    s_ref[...] = jnp.dot(x, ws_ref[...],
                         preferred_element_type=jnp.float32) + b_ref[...]
    g_ref[...] = jnp.dot(x, wn_ref[...], preferred_element_type=jnp.float32)


def _tc3_body(s2_ref, p_ref, dg_ref, o_ref):
    psum = p_ref[0] + p_ref[1]
    deg = dg_ref[0] + dg_ref[1]
    inv = 1.0 / jnp.maximum(deg, 1.0)
    o_ref[...] = s2_ref[...] + psum * inv


def _row_spec():
    return pl.BlockSpec((_TCB, D), lambda i: (i, 0))


_W_SPEC = pl.BlockSpec((D, D), lambda i: (0, 0))
_B_SPEC = pl.BlockSpec((1, D), lambda i: (0, 0))
_P_SPEC = pl.BlockSpec((NC, _TCB, D), lambda i: (0, i, 0))
_DG_SPEC = pl.BlockSpec((NC, _TCB, 1), lambda i: (0, i, 0))

_tc1 = pl.pallas_call(
    _tc1_body, grid=(_TCG,),
    in_specs=[_row_spec(), _W_SPEC, _W_SPEC, _B_SPEC],
    out_specs=[_row_spec(), _row_spec()],
    out_shape=[jax.ShapeDtypeStruct((N, D), jnp.float32)] * 2,
)

_tc2 = pl.pallas_call(
    _tc2_body, grid=(_TCG,),
    in_specs=[_row_spec(), _P_SPEC, _DG_SPEC, _W_SPEC, _W_SPEC, _B_SPEC],
    out_specs=[_row_spec(), _row_spec()],
    out_shape=[jax.ShapeDtypeStruct((N, D), jnp.float32)] * 2,
)

_tc3 = pl.pallas_call(
    _tc3_body, grid=(_TCG,),
    in_specs=[_row_spec(), _P_SPEC, _DG_SPEC],
    out_specs=_row_spec(),
    out_shape=jax.ShapeDtypeStruct((N, D), jnp.float32),
)


def kernel(h, edge_index, W_self1, W_neigh1, b1, W_self2, W_neigh2, b2):
    src = edge_index[0].astype(jnp.int32)
    dst = edge_index[1].astype(jnp.int32)
    trash = TRASH + jnp.arange(SLAB - EPT, dtype=jnp.int32) % (NPAD - N)
    pad_words = jnp.broadcast_to(trash << 16, (NW, SLAB - EPT))
    packed = jnp.concatenate(
        [(src | (dst << 16)).reshape(NW, EPT), pad_words], axis=1)
    b1r = b1.reshape(1, D)
    b2r = b2.reshape(1, D)

    sc1 = _make_sc(True)
    sc2 = _make_sc(False)

    s1, g1 = _tc1(h, W_self1, W_neigh1, b1r)
    p1_flat, degp = sc1(g1, packed)
    degp = degp.reshape(NC, NPAD, 1)
    p1 = p1_flat.reshape(NC, NPAD, D)
    s2, g2 = _tc2(s1, p1, degp, W_self2, W_neigh2, b2r)
    p2_flat, _ = sc2(g2, packed)
    p2 = p2_flat.reshape(NC, NPAD, D)
    return _tc3(s2, p2, degp)


# speedup vs baseline: 3.4139x; 3.4139x over previous
import functools

import jax
import jax.numpy as jnp
from jax import lax
from jax.experimental import pallas as pl
from jax.experimental.pallas import tpu as pltpu
from jax.experimental.pallas import tpu_sc as plsc

N = 10000
E = 320000
D = 128
NPAD = 10240
NC = 2
NS = 16
NW = NC * NS
EPT = E // NW
CH = 80
NCHUNK = EPT // CH
RPT = NPAD // NS


def _sc_body(compute_deg, g_hbm, pk_hbm, acc_out, deg_out,
             acc_sp, deg_sp, packed, rows_a, rows_b, rows_c,
             sa_src, sa_dst, sb_src, sb_dst, sc_src, sc_dst,
             ones_v, zrow_v, ga, gb, gc, sa, sb, sc_, dsem):
    c = lax.axis_index("c")
    s = lax.axis_index("s")
    wid = c * NS + s

    pltpu.sync_copy(pk_hbm.at[wid], packed)

    def _zero_row(j, carry):
        for i in range(D // 16):
            rows_a[j, pl.ds(i * 16, 16)] = jnp.zeros((16,), jnp.float32)
        return carry
    lax.fori_loop(0, CH, _zero_row, 0, unroll=False)

    def _zrow16(i, carry):
        zrow_v[pl.ds(i * 16, 16)] = jnp.zeros((16,), jnp.float32)
        return carry
    lax.fori_loop(0, RPT // 16, _zrow16, 0, unroll=False)
    if compute_deg:
        def _ones16(i, carry):
            ones_v[pl.ds(i * 16, 16)] = jnp.ones((16,), jnp.float32)
            return carry
        lax.fori_loop(0, CH // 16, _ones16, 0, unroll=False)

    off = 0
    while off < RPT:
        nrows = min(CH, RPT - off)
        pltpu.sync_copy(rows_a.at[pl.ds(0, nrows)],
                        acc_sp.at[pl.ds(s * RPT + off, nrows)])
        off += nrows
    if compute_deg:
        pltpu.sync_copy(zrow_v, deg_sp.at[pl.ds(s * RPT, RPT)])

    def unpack(k, st_src, st_dst):
        for i in range(CH // 16):
            w = packed[pl.ds(k * CH + i * 16, 16)]
            st_src[pl.ds(i * 16, 16)] = w & 0xFFFF
            st_dst[pl.ds(i * 16, 16)] = w >> 16

    def gather(st_src, buf, gsem):
        pltpu.async_copy(g_hbm.at[st_src], buf, gsem)

    def gather_wait(st_src, buf, gsem):
        pltpu.make_async_copy(g_hbm.at[st_src], buf, gsem).wait()

    def scat(st_dst, buf, ssem):
        pltpu.async_copy(buf, acc_sp.at[st_dst], ssem, add=True)
        if compute_deg:
            pltpu.async_copy(ones_v, deg_sp.at[st_dst], dsem, add=True)

    def scat_wait(st_dst, buf, ssem):
        pltpu.make_async_copy(buf, acc_sp.at[st_dst], ssem).wait()
        if compute_deg:
            pltpu.make_async_copy(ones_v, deg_sp.at[st_dst], dsem).wait()

    unpack(0, sa_src, sa_dst)
    unpack(1, sb_src, sb_dst)
    gather(sa_src, rows_a, ga)
    gather(sb_src, rows_b, gb)
    plsc.subcore_barrier()

    def body(t, carry):
        k0 = 3 * t
        gather_wait(sa_src, rows_a, ga)
        scat(sa_dst, rows_a, sa)

        @pl.when(t > 0)
        def _():
            scat_wait(sc_dst, rows_c, sc_)
        unpack(k0 + 2, sc_src, sc_dst)
        gather(sc_src, rows_c, gc)

        gather_wait(sb_src, rows_b, gb)
        scat(sb_dst, rows_b, sb)

        scat_wait(sa_dst, rows_a, sa)
        unpack(k0 + 3, sa_src, sa_dst)
        gather(sa_src, rows_a, ga)

        gather_wait(sc_src, rows_c, gc)
        scat(sc_dst, rows_c, sc_)

        scat_wait(sb_dst, rows_b, sb)
        unpack(k0 + 4, sb_src, sb_dst)
        gather(sb_src, rows_b, gb)
        return carry

    lax.fori_loop(0, (NCHUNK - 2) // 3, body, 0, unroll=False)

    gather_wait(sa_src, rows_a, ga)
    scat(sa_dst, rows_a, sa)
    scat_wait(sc_dst, rows_c, sc_)
    gather_wait(sb_src, rows_b, gb)
    scat(sb_dst, rows_b, sb)
    scat_wait(sa_dst, rows_a, sa)
    scat_wait(sb_dst, rows_b, sb)
    plsc.subcore_barrier()

    pltpu.sync_copy(acc_sp.at[pl.ds(s * RPT, RPT)],
                    acc_out.at[pl.ds(c * NPAD + s * RPT, RPT)])
    if compute_deg:
        pltpu.sync_copy(deg_sp.at[pl.ds(s * RPT, RPT)],
                        deg_out.at[c, pl.ds(s * RPT, RPT)])


def _make_sc(compute_deg):
    out_type = (jax.ShapeDtypeStruct((NC * NPAD, D), jnp.float32),
                jax.ShapeDtypeStruct((NC, NPAD), jnp.float32))
    scratch = [
        pltpu.VMEM_SHARED((NPAD, D), jnp.float32),
        pltpu.VMEM_SHARED((NPAD,), jnp.float32),
        pltpu.VMEM((EPT,), jnp.int32),
        pltpu.VMEM((CH, D), jnp.float32),
        pltpu.VMEM((CH, D), jnp.float32),
        pltpu.VMEM((CH, D), jnp.float32),
        pltpu.VMEM((CH,), jnp.int32),
        pltpu.VMEM((CH,), jnp.int32),
        pltpu.VMEM((CH,), jnp.int32),
        pltpu.VMEM((CH,), jnp.int32),
        pltpu.VMEM((CH,), jnp.int32),
        pltpu.VMEM((CH,), jnp.int32),
        pltpu.VMEM((CH,), jnp.float32),
        pltpu.VMEM((RPT,), jnp.float32),
        pltpu.SemaphoreType.DMA,
        pltpu.SemaphoreType.DMA,
        pltpu.SemaphoreType.DMA,
        pltpu.SemaphoreType.DMA,
        pltpu.SemaphoreType.DMA,
        pltpu.SemaphoreType.DMA,
        pltpu.SemaphoreType.DMA,
    ]
    mesh = plsc.VectorSubcoreMesh(core_axis_name="c", subcore_axis_name="s",
                                  num_cores=NC, num_subcores=NS)
    return pl.kernel(functools.partial(_sc_body, compute_deg),
                     out_type=out_type, mesh=mesh, scratch_types=scratch)


_TCB = 1024
_TCG = NPAD // _TCB


def _tc1_body(h_ref, ws_ref, wn_ref, b_ref, s_ref, g_ref):
    hb = h_ref[...]
    s_ref[...] = jnp.dot(hb, ws_ref[...],
                         preferred_element_type=jnp.float32) + b_ref[...]
    g_ref[...] = jnp.dot(hb, wn_ref[...], preferred_element_type=jnp.float32)


def _tc2_body(s1_ref, p_ref, dg_ref, ws_ref, wn_ref, b_ref, s_ref, g_ref):
    psum = p_ref[0] + p_ref[1]
    deg = dg_ref[0] + dg_ref[1]
    inv = 1.0 / jnp.maximum(deg, 1.0)
    x = jnp.maximum(s1_ref[...] + psum * inv[:, None], 0.0)
    s_ref[...] = jnp.dot(x, ws_ref[...],
                         preferred_element_type=jnp.float32) + b_ref[...]
    g_ref[...] = jnp.dot(x, wn_ref[...], preferred_element_type=jnp.float32)


def _tc3_body(s2_ref, p_ref, dg_ref, o_ref):
    psum = p_ref[0] + p_ref[1]
    deg = dg_ref[0] + dg_ref[1]
    inv = 1.0 / jnp.maximum(deg, 1.0)
    o_ref[...] = s2_ref[...] + psum * inv[:, None]


def _row_spec():
    return pl.BlockSpec((_TCB, D), lambda i: (i, 0))


_W_SPEC = pl.BlockSpec((D, D), lambda i: (0, 0))
_B_SPEC = pl.BlockSpec((1, D), lambda i: (0, 0))
_P_SPEC = pl.BlockSpec((NC, _TCB, D), lambda i: (0, i, 0))
_DG_SPEC = pl.BlockSpec((NC, _TCB), lambda i: (0, i))

_tc1 = pl.pallas_call(
    _tc1_body, grid=(_TCG,),
    in_specs=[_row_spec(), _W_SPEC, _W_SPEC, _B_SPEC],
    out_specs=[_row_spec(), _row_spec()],
    out_shape=[jax.ShapeDtypeStruct((NPAD, D), jnp.float32)] * 2,
)

_tc2 = pl.pallas_call(
    _tc2_body, grid=(_TCG,),
    in_specs=[_row_spec(), _P_SPEC, _DG_SPEC, _W_SPEC, _W_SPEC, _B_SPEC],
    out_specs=[_row_spec(), _row_spec()],
    out_shape=[jax.ShapeDtypeStruct((NPAD, D), jnp.float32)] * 2,
)

_tc3 = pl.pallas_call(
    _tc3_body, grid=(_TCG,),
    in_specs=[_row_spec(), _P_SPEC, _DG_SPEC],
    out_specs=_row_spec(),
    out_shape=jax.ShapeDtypeStruct((NPAD, D), jnp.float32),
)


def kernel(h, edge_index, W_self1, W_neigh1, b1, W_self2, W_neigh2, b2):
    src = edge_index[0].astype(jnp.int32)
    dst = edge_index[1].astype(jnp.int32)
    packed = (src | (dst << 16)).reshape(NW, EPT)
    h_pad = jnp.pad(h, ((0, NPAD - N), (0, 0)))
    b1r = b1.reshape(1, D)
    b2r = b2.reshape(1, D)

    sc1 = _make_sc(True)
    sc2 = _make_sc(False)

    s1, g1 = _tc1(h_pad, W_self1, W_neigh1, b1r)
    p1_flat, degp = sc1(g1, packed)
    p1 = p1_flat.reshape(NC, NPAD, D)
    s2, g2 = _tc2(s1, p1, degp, W_self2, W_neigh2, b2r)
    p2_flat, _ = sc2(g2, packed)
    p2 = p2_flat.reshape(NC, NPAD, D)
    out = _tc3(s2, p2, degp)
    return out[:N]
